# baseline (speedup 1.0000x reference)
import math

import jax
import jax.numpy as jnp
from jax import lax
from jax.experimental import pallas as pl
from jax.experimental.pallas import tpu as pltpu
from jax.experimental.pallas import tpu_sc as plsc

_D = 64
_SCALE = math.sqrt(_D)
_NC, _NS = 2, 16
_NW = _NC * _NS
_IDX_GRP = 128
_GRPS = 4
_CHUNK = _IDX_GRP * _GRPS


def _build(n_rows):
    rows_per_w = n_rows // _NW
    n_chunks = rows_per_w // _CHUNK
    grps_per_w = rows_per_w // _IDX_GRP
    mesh = plsc.VectorSubcoreMesh(
        core_axis_name="c", subcore_axis_name="s",
        num_cores=_NC, num_subcores=_NS)

    def body(idx_hbm, table_hbm, out_hbm, idx_v, rows_v, sem):
        wid = lax.axis_index("s") * _NC + lax.axis_index("c")
        base_row = wid * rows_per_w
        base_grp = wid * grps_per_w

        def chunk_body(g, carry):
            row0 = base_row + g * _CHUNK
            pltpu.sync_copy(idx_hbm.at[pl.ds(base_grp + g * _GRPS, _GRPS)],
                            idx_v)
            cps = [
                pltpu.async_copy(
                    table_hbm.at[idx_v.at[j]],
                    rows_v.at[pl.ds(j * _IDX_GRP, _IDX_GRP)],
                    sem)
                for j in range(_GRPS)
            ]
            for cp in cps:
                cp.wait()

            def scale_body(r, c):
                for k in range(_D // 16):
                    sl = pl.ds(k * 16, 16)
                    rows_v[r, sl] = rows_v[r, sl] * _SCALE
                return c

            lax.fori_loop(0, _CHUNK, scale_body, 0)
            pltpu.sync_copy(rows_v, out_hbm.at[pl.ds(row0, _CHUNK)])
            return carry

        lax.fori_loop(0, n_chunks, chunk_body, 0)

    return pl.kernel(
        body,
        out_type=jax.ShapeDtypeStruct((n_rows, _D), jnp.float32),
        mesh=mesh,
        compiler_params=pltpu.CompilerParams(use_tc_tiling_on_sc=False),
        scratch_types=[
            pltpu.VMEM((_GRPS, _IDX_GRP), jnp.int32),
            pltpu.VMEM((_CHUNK, _D), jnp.float32),
            pltpu.SemaphoreType.DMA,
        ],
    )


def kernel(token_ids_batch, embeddings_table):
    b, s = token_ids_batch.shape
    n = b * s
    idx2d = token_ids_batch.astype(jnp.int32).reshape(n // _IDX_GRP, _IDX_GRP)
    out = _build(n)(idx2d, embeddings_table)
    return out.reshape(b, s, _D)


# speedup vs baseline: 1.1314x; 1.1314x over previous
import math

import jax
import jax.numpy as jnp
from jax import lax
from jax.experimental import pallas as pl
from jax.experimental.pallas import tpu as pltpu
from jax.experimental.pallas import tpu_sc as plsc

_D = 64
_SCALE = math.sqrt(_D)
_NC, _NS = 2, 16
_NW = _NC * _NS
_NB = 4


def _build(batch, seq):
    bpw = batch // _NW
    n_chunks = bpw // _NB
    seq_lo = min(seq, 128)
    seq_hi = seq - seq_lo
    mesh = plsc.VectorSubcoreMesh(
        core_axis_name="c", subcore_axis_name="s",
        num_cores=_NC, num_subcores=_NS)

    def fire(idx_hbm, table_hbm, idx_v, rows_v, sem, bb):
        pltpu.sync_copy(idx_hbm.at[pl.ds(bb, _NB)], idx_v)
        for r in range(_NB):
            pltpu.async_copy(
                table_hbm.at[idx_v.at[r, pl.ds(0, seq_lo)]],
                rows_v.at[r, pl.ds(0, seq_lo)], sem)
            if seq_hi:
                pltpu.async_copy(
                    table_hbm.at[idx_v.at[r, pl.ds(seq_lo, seq_hi)]],
                    rows_v.at[r, pl.ds(seq_lo, seq_hi)], sem)

    def body(idx_hbm, table_hbm, out_hbm, idx0, idx1, rows0, rows1,
             sem0, sem1):
        wid = lax.axis_index("s") * _NC + lax.axis_index("c")
        b0 = wid * bpw

        fire(idx_hbm, table_hbm, idx0, rows0, sem0, b0)

        def step(idx_v, rows_v, sem, idx_n, rows_n, sem_n, g):
            bb = b0 + g * _NB
            pltpu.make_async_copy(out_hbm.at[pl.ds(bb, _NB)], rows_v,
                                  sem).wait()

            @pl.when(g + 1 < n_chunks)
            def _():
                fire(idx_hbm, table_hbm, idx_n, rows_n, sem_n, bb + _NB)

            for r in range(_NB):
                def scale(s4, c):
                    for ds_ in range(4):
                        for k in range(_D // 16):
                            sl = pl.ds(k * 16, 16)
                            rows_v[r, s4 * 4 + ds_, sl] = (
                                rows_v[r, s4 * 4 + ds_, sl] * _SCALE)
                    return c
                lax.fori_loop(0, seq // 4, scale, 0)
            pltpu.sync_copy(rows_v, out_hbm.at[pl.ds(bb, _NB)])

        def loop(g2, carry):
            step(idx0, rows0, sem0, idx1, rows1, sem1, 2 * g2)
            step(idx1, rows1, sem1, idx0, rows0, sem0, 2 * g2 + 1)
            return carry

        lax.fori_loop(0, n_chunks // 2, loop, 0)

    return pl.kernel(
        body,
        out_type=jax.ShapeDtypeStruct((batch, seq, _D), jnp.float32),
        mesh=mesh,
        compiler_params=pltpu.CompilerParams(use_tc_tiling_on_sc=False),
        scratch_types=[
            pltpu.VMEM((_NB, seq), jnp.int32),
            pltpu.VMEM((_NB, seq), jnp.int32),
            pltpu.VMEM((_NB, seq, _D), jnp.float32),
            pltpu.VMEM((_NB, seq, _D), jnp.float32),
            pltpu.SemaphoreType.DMA,
            pltpu.SemaphoreType.DMA,
        ],
    )


def kernel(token_ids_batch, embeddings_table):
    b, s = token_ids_batch.shape
    return _build(b, s)(token_ids_batch.astype(jnp.int32), embeddings_table)
